# baseline (speedup 1.0000x reference)
import functools

import jax
import jax.numpy as jnp
from jax import lax
from jax.experimental import pallas as pl
from jax.experimental.pallas import tpu as pltpu

D_MODEL = 1024
HIDDEN = 2048
N_EXPERTS = 8
TOP_K = 2

TM = 512
TH = 512
NH = HIDDEN // TH

_INTERPRET = False


def _grouped_swiglu_kernel(meta_ref, x_ref, wg_ref, wu_ref, wd_ref, o_ref):
    i = pl.program_id(0)
    h = pl.program_id(1)
    xb = x_ref[...].astype(jnp.bfloat16)
    a = jnp.dot(xb, wg_ref[0], preferred_element_type=jnp.float32)
    b = jnp.dot(xb, wu_ref[0], preferred_element_type=jnp.float32)
    g = ((a * jax.nn.sigmoid(a)) * b).astype(jnp.bfloat16)
    contrib = jnp.dot(g, wd_ref[0], preferred_element_type=jnp.float32)

    @pl.when(h == 0)
    def _():
        o_ref[...] = contrib

    @pl.when(h != 0)
    def _():
        o_ref[...] = o_ref[...] + contrib

    @pl.when(h == NH - 1)
    def _():
        rem = meta_ref[1, i]
        rows = lax.broadcasted_iota(jnp.int32, (TM, 1), 0)
        o_ref[...] = jnp.where(rows < rem, o_ref[...], 0.0)


def _grouped_swiglu(meta, xs, Wg, Wu, Wd, nt):
    np_rows = nt * TM
    grid_spec = pltpu.PrefetchScalarGridSpec(
        num_scalar_prefetch=1,
        grid=(nt, NH),
        in_specs=[
            pl.BlockSpec((TM, D_MODEL), lambda i, h, m: (i, 0)),
            pl.BlockSpec((1, D_MODEL, TH), lambda i, h, m: (m[0, i], 0, h)),
            pl.BlockSpec((1, D_MODEL, TH), lambda i, h, m: (m[0, i], 0, h)),
            pl.BlockSpec((1, TH, D_MODEL), lambda i, h, m: (m[0, i], h, 0)),
        ],
        out_specs=pl.BlockSpec((TM, D_MODEL), lambda i, h, m: (i, 0)),
    )
    return pl.pallas_call(
        _grouped_swiglu_kernel,
        grid_spec=grid_spec,
        out_shape=jax.ShapeDtypeStruct((np_rows, D_MODEL), jnp.float32),
        compiler_params=pltpu.CompilerParams(
            dimension_semantics=("arbitrary", "arbitrary")),
        interpret=_INTERPRET,
    )(meta, xs, Wg, Wu, Wd)


def _shared_combine_kernel(x_ref, wg_ref, wu_ref, wd_ref, b0_ref, b1_ref,
                           g0_ref, g1_ref, o_ref):
    h = pl.program_id(1)
    xb = x_ref[...].astype(jnp.bfloat16)
    a = jnp.dot(xb, wg_ref[...], preferred_element_type=jnp.float32)
    b = jnp.dot(xb, wu_ref[...], preferred_element_type=jnp.float32)
    g = ((a * jax.nn.sigmoid(a)) * b).astype(jnp.bfloat16)
    contrib = jnp.dot(g, wd_ref[...], preferred_element_type=jnp.float32)

    @pl.when(h == 0)
    def _():
        o_ref[...] = contrib

    @pl.when(h != 0)
    def _():
        o_ref[...] = o_ref[...] + contrib

    @pl.when(h == NH - 1)
    def _():
        o_ref[...] = (o_ref[...]
                      + g0_ref[:, :1] * b0_ref[...]
                      + g1_ref[:, :1] * b1_ref[...])


def _shared_combine(x_flat, sWg, sWu, sWd, buf0, buf1, g0, g1):
    t = x_flat.shape[0]
    grid = (t // TM, NH)
    return pl.pallas_call(
        _shared_combine_kernel,
        grid=grid,
        in_specs=[
            pl.BlockSpec((TM, D_MODEL), lambda i, h: (i, 0)),
            pl.BlockSpec((D_MODEL, TH), lambda i, h: (0, h)),
            pl.BlockSpec((D_MODEL, TH), lambda i, h: (0, h)),
            pl.BlockSpec((TH, D_MODEL), lambda i, h: (h, 0)),
            pl.BlockSpec((TM, D_MODEL), lambda i, h: (i, 0)),
            pl.BlockSpec((TM, D_MODEL), lambda i, h: (i, 0)),
            pl.BlockSpec((TM, 128), lambda i, h: (i, 0)),
            pl.BlockSpec((TM, 128), lambda i, h: (i, 0)),
        ],
        out_specs=pl.BlockSpec((TM, D_MODEL), lambda i, h: (i, 0)),
        out_shape=jax.ShapeDtypeStruct((t, D_MODEL), jnp.float32),
        compiler_params=pltpu.CompilerParams(
            dimension_semantics=("arbitrary", "arbitrary")),
        interpret=_INTERPRET,
    )(x_flat, sWg, sWu, sWd, buf0, buf1, g0, g1)


def kernel(x, Wr, Wg, Wu, Wd, sWg, sWu, sWd):
    b, s, d = x.shape
    t = b * s
    a_total = t * TOP_K
    nt = a_total // TM + N_EXPERTS
    np_rows = nt * TM
    x_flat = x.reshape(t, d)
    Wg = Wg.astype(jnp.bfloat16)
    Wu = Wu.astype(jnp.bfloat16)
    Wd = Wd.astype(jnp.bfloat16)
    sWg = sWg.astype(jnp.bfloat16)
    sWu = sWu.astype(jnp.bfloat16)
    sWd = sWd.astype(jnp.bfloat16)

    logits = x_flat @ Wr
    top_logits, top_idx = lax.top_k(logits, TOP_K)
    gate = jax.nn.softmax(top_logits, axis=-1)

    e_flat = jnp.concatenate([top_idx[:, 0], top_idx[:, 1]])
    onehot = (e_flat[:, None] == jnp.arange(N_EXPERTS)[None, :]).astype(jnp.int32)
    counts = onehot.sum(axis=0)
    nt_e = (counts + TM - 1) // TM
    cum_nt = jnp.cumsum(nt_e)
    first_tile = cum_nt - nt_e
    nt_total = cum_nt[-1]
    group_start = first_tile * TM
    rank = jnp.sum((jnp.cumsum(onehot, axis=0) - onehot) * onehot, axis=1)
    pos = group_start[e_flat] + rank
    tok = jnp.concatenate([jnp.arange(t, dtype=jnp.int32)] * 2)
    sorted_tok = jnp.zeros((np_rows,), jnp.int32).at[pos].set(tok)

    ti = jnp.arange(nt, dtype=jnp.int32)
    tile_e = jnp.clip(jnp.searchsorted(cum_nt, ti, side='right'), 0,
                      N_EXPERTS - 1).astype(jnp.int32)
    rem = counts[tile_e] - (ti - first_tile[tile_e]) * TM
    rem = jnp.where(ti < nt_total, rem, 0)
    meta = jnp.stack([tile_e, rem]).astype(jnp.int32)

    xs = jnp.take(x_flat, sorted_tok, axis=0)

    y = _grouped_swiglu(meta, xs, Wg, Wu, Wd, nt)

    buf = jnp.take(y, pos, axis=0)
    g0 = jnp.broadcast_to(gate[:, 0:1], (t, 128))
    g1 = jnp.broadcast_to(gate[:, 1:2], (t, 128))

    out = _shared_combine(x_flat, sWg, sWu, sWd, buf[:t], buf[t:], g0, g1)
    return out.reshape(b, s, d)


# speedup vs baseline: 1.1508x; 1.1508x over previous
import functools

import jax
import jax.numpy as jnp
from jax import lax
from jax.experimental import pallas as pl
from jax.experimental.pallas import tpu as pltpu

D_MODEL = 1024
HIDDEN = 2048
N_EXPERTS = 8
TOP_K = 2

TM = 512

_INTERPRET = False


def _grouped_swiglu_kernel(meta_ref, x_ref, wg_ref, wu_ref, wd_ref, o_ref):
    i = pl.program_id(0)
    xb = x_ref[...].astype(jnp.bfloat16)
    a = jnp.dot(xb, wg_ref[0], preferred_element_type=jnp.float32)
    b = jnp.dot(xb, wu_ref[0], preferred_element_type=jnp.float32)
    g = ((a * jax.nn.sigmoid(a)) * b).astype(jnp.bfloat16)
    contrib = jnp.dot(g, wd_ref[0], preferred_element_type=jnp.float32)
    rem = meta_ref[1, i]
    rows = lax.broadcasted_iota(jnp.int32, (TM, 1), 0)
    o_ref[...] = jnp.where(rows < rem, contrib, 0.0)


def _grouped_swiglu(meta, xs, Wg, Wu, Wd, nt):
    np_rows = nt * TM
    grid_spec = pltpu.PrefetchScalarGridSpec(
        num_scalar_prefetch=1,
        grid=(nt,),
        in_specs=[
            pl.BlockSpec((TM, D_MODEL), lambda i, m: (i, 0)),
            pl.BlockSpec((1, D_MODEL, HIDDEN), lambda i, m: (m[0, i], 0, 0)),
            pl.BlockSpec((1, D_MODEL, HIDDEN), lambda i, m: (m[0, i], 0, 0)),
            pl.BlockSpec((1, HIDDEN, D_MODEL), lambda i, m: (m[0, i], 0, 0)),
        ],
        out_specs=pl.BlockSpec((TM, D_MODEL), lambda i, m: (i, 0)),
    )
    return pl.pallas_call(
        _grouped_swiglu_kernel,
        grid_spec=grid_spec,
        out_shape=jax.ShapeDtypeStruct((np_rows, D_MODEL), jnp.float32),
        compiler_params=pltpu.CompilerParams(
            dimension_semantics=("arbitrary",)),
        interpret=_INTERPRET,
    )(meta, xs, Wg, Wu, Wd)


def _shared_combine_kernel(x_ref, wg_ref, wu_ref, wd_ref, b0_ref, b1_ref,
                           g0_ref, g1_ref, o_ref):
    xb = x_ref[...].astype(jnp.bfloat16)
    a = jnp.dot(xb, wg_ref[...], preferred_element_type=jnp.float32)
    b = jnp.dot(xb, wu_ref[...], preferred_element_type=jnp.float32)
    g = ((a * jax.nn.sigmoid(a)) * b).astype(jnp.bfloat16)
    contrib = jnp.dot(g, wd_ref[...], preferred_element_type=jnp.float32)
    o_ref[...] = (contrib
                  + g0_ref[:, :1] * b0_ref[...]
                  + g1_ref[:, :1] * b1_ref[...])


def _shared_combine(x_flat, sWg, sWu, sWd, buf0, buf1, g0, g1):
    t = x_flat.shape[0]
    grid = (t // TM,)
    return pl.pallas_call(
        _shared_combine_kernel,
        grid=grid,
        in_specs=[
            pl.BlockSpec((TM, D_MODEL), lambda i: (i, 0)),
            pl.BlockSpec((D_MODEL, HIDDEN), lambda i: (0, 0)),
            pl.BlockSpec((D_MODEL, HIDDEN), lambda i: (0, 0)),
            pl.BlockSpec((HIDDEN, D_MODEL), lambda i: (0, 0)),
            pl.BlockSpec((TM, D_MODEL), lambda i: (i, 0)),
            pl.BlockSpec((TM, D_MODEL), lambda i: (i, 0)),
            pl.BlockSpec((TM, 128), lambda i: (i, 0)),
            pl.BlockSpec((TM, 128), lambda i: (i, 0)),
        ],
        out_specs=pl.BlockSpec((TM, D_MODEL), lambda i: (i, 0)),
        out_shape=jax.ShapeDtypeStruct((t, D_MODEL), jnp.float32),
        compiler_params=pltpu.CompilerParams(
            dimension_semantics=("arbitrary",)),
        interpret=_INTERPRET,
    )(x_flat, sWg, sWu, sWd, buf0, buf1, g0, g1)


def kernel(x, Wr, Wg, Wu, Wd, sWg, sWu, sWd):
    b, s, d = x.shape
    t = b * s
    a_total = t * TOP_K
    nt = a_total // TM + N_EXPERTS
    np_rows = nt * TM
    x_flat = x.reshape(t, d)
    Wg = Wg.astype(jnp.bfloat16)
    Wu = Wu.astype(jnp.bfloat16)
    Wd = Wd.astype(jnp.bfloat16)
    sWg = sWg.astype(jnp.bfloat16)
    sWu = sWu.astype(jnp.bfloat16)
    sWd = sWd.astype(jnp.bfloat16)

    logits = x_flat @ Wr
    top_logits, top_idx = lax.top_k(logits, TOP_K)
    gate = jax.nn.softmax(top_logits, axis=-1)

    e_flat = jnp.concatenate([top_idx[:, 0], top_idx[:, 1]])
    onehot = (e_flat[:, None] == jnp.arange(N_EXPERTS)[None, :]).astype(jnp.int32)
    counts = onehot.sum(axis=0)
    nt_e = (counts + TM - 1) // TM
    cum_nt = jnp.cumsum(nt_e)
    first_tile = cum_nt - nt_e
    nt_total = cum_nt[-1]
    group_start = first_tile * TM
    rank = jnp.sum((jnp.cumsum(onehot, axis=0) - onehot) * onehot, axis=1)
    pos = group_start[e_flat] + rank
    tok = jnp.concatenate([jnp.arange(t, dtype=jnp.int32)] * 2)
    sorted_tok = jnp.zeros((np_rows,), jnp.int32).at[pos].set(tok)

    ti = jnp.arange(nt, dtype=jnp.int32)
    tile_e = jnp.clip(jnp.searchsorted(cum_nt, ti, side='right'), 0,
                      N_EXPERTS - 1).astype(jnp.int32)
    rem = counts[tile_e] - (ti - first_tile[tile_e]) * TM
    rem = jnp.where(ti < nt_total, rem, 0)
    meta = jnp.stack([tile_e, rem]).astype(jnp.int32)

    xs = jnp.take(x_flat, sorted_tok, axis=0)

    y = _grouped_swiglu(meta, xs, Wg, Wu, Wd, nt)

    buf = jnp.take(y, pos, axis=0)
    g0 = jnp.broadcast_to(gate[:, 0:1], (t, 128))
    g1 = jnp.broadcast_to(gate[:, 1:2], (t, 128))

    out = _shared_combine(x_flat, sWg, sWu, sWd, buf[:t], buf[t:], g0, g1)
    return out.reshape(b, s, d)
